# baseline (speedup 1.0000x reference)
import functools

import jax
import jax.numpy as jnp
import numpy as np
from jax import lax
from jax.experimental import pallas as pl
from jax.experimental.pallas import tpu as pltpu
from jax.experimental.pallas import tpu_sc as plsc

_MATCH = 4
_L = 16

_GATHER_DNUMS = lax.GatherDimensionNumbers(
    offset_dims=(), collapsed_slice_dims=(0,), start_index_map=(0,)
)


def _lane_shuffle(x, perm):
    return lax.gather(
        x,
        perm[:, None],
        _GATHER_DNUMS,
        slice_sizes=(1,),
        mode=lax.GatherScatterMode.PROMISE_IN_BOUNDS,
    )


def _cxcywh(x):
    x0, y0, x1, y1 = jnp.split(x, 4, axis=-1)
    return jnp.concatenate(
        [(x0 + x1) / 2.0, (y0 + y1) / 2.0, x1 - x0, y1 - y0], axis=-1
    )


def _sc_body(nq, ncol, anch_hbm, tgt_hbm, out_hbm, coords_v, tgt_v, out_v):
    nchunk = nq // _L
    cid = lax.axis_index("c")
    sid = lax.axis_index("s")
    wid = sid * 2 + cid
    b = wid // 4
    sub = wid % 4

    pltpu.sync_copy(anch_hbm, coords_v)
    pltpu.sync_copy(tgt_hbm.at[b, sub], tgt_v)

    lane = lax.broadcasted_iota(jnp.int32, (_L,), 0)
    inf = jnp.float32(jnp.inf)
    trow0 = tgt_v[0]
    trow1 = tgt_v[1]
    trow2 = tgt_v[2]
    trow3 = tgt_v[3]

    def per_target(c, acc):
        t0 = trow0[c]
        t1 = trow1[c]
        t2 = trow2[c]
        t3 = trow3[c]

        def scan_chunk(i, carry):
            m0, m1, m2, m3, j0, j1, j2, j3, idx = carry
            s = pl.ds(i * _L, _L)
            v = jnp.abs(coords_v[0, s] - t0)
            v = v + jnp.abs(coords_v[1, s] - t1)
            v = v + jnp.abs(coords_v[2, s] - t2)
            v = v + jnp.abs(coords_v[3, s] - t3)
            b0 = v < m0
            b1 = v < m1
            b2 = v < m2
            b3 = v < m3
            m3 = jnp.where(b3, jnp.where(b2, m2, v), m3)
            j3 = jnp.where(b3, jnp.where(b2, j2, idx), j3)
            m2 = jnp.where(b2, jnp.where(b1, m1, v), m2)
            j2 = jnp.where(b2, jnp.where(b1, j1, idx), j2)
            m1 = jnp.where(b1, jnp.where(b0, m0, v), m1)
            j1 = jnp.where(b1, jnp.where(b0, j0, idx), j1)
            m0 = jnp.where(b0, v, m0)
            j0 = jnp.where(b0, idx, j0)
            return (m0, m1, m2, m3, j0, j1, j2, j3, idx + _L)

        finf = jnp.full((_L,), inf)
        zi = jnp.zeros((_L,), jnp.int32)
        m0, m1, m2, m3, j0, j1, j2, j3, _ = lax.fori_loop(
            0, nchunk, scan_chunk,
            (finf, finf, finf, finf, zi, zi, zi, zi, lane),
        )

        for r in range(_MATCH):
            gv, gj = m0, j0
            for off in (8, 4, 2, 1):
                perm = lane ^ off
                pv = _lane_shuffle(gv, perm)
                pj = _lane_shuffle(gj, perm)
                pick = (pv < gv) | ((pv == gv) & (pj < gj))
                gv = jnp.where(pick, pv, gv)
                gj = jnp.where(pick, pj, gj)
            hit = (m0 == gv) & (j0 == gj)
            m0 = jnp.where(hit, m1, m0)
            j0 = jnp.where(hit, j1, j0)
            m1 = jnp.where(hit, m2, m1)
            j1 = jnp.where(hit, j2, j1)
            m2 = jnp.where(hit, m3, m2)
            j2 = jnp.where(hit, j3, j2)
            m3 = jnp.where(hit, inf, m3)
            acc = tuple(
                jnp.where(lane == c, gj, acc[q]) if q == r else acc[q]
                for q in range(_MATCH)
            )
        return acc

    zacc = jnp.zeros((_L,), jnp.int32)
    acc = (zacc,) * _MATCH
    for c in range(ncol):
        acc = per_target(c, acc)
    for r in range(_MATCH):
        out_v[r] = acc[r]
    pltpu.sync_copy(out_v, out_hbm.at[b, sub])


def _tc_body(nq, nt, nb, nca, pred_ref, anch_ref, tgt_ref, out_ref):
    nrow = nt + nca
    t = tgt_ref[0]
    a0 = anch_ref[0:1, :]
    a1 = anch_ref[1:2, :]
    a2 = anch_ref[2:3, :]
    a3 = anch_ref[3:4, :]
    drows = []
    for i in range(nb):
        x0 = pred_ref[0, 4 * i + 0:4 * i + 1, :]
        y0 = pred_ref[0, 4 * i + 1:4 * i + 2, :]
        x1 = pred_ref[0, 4 * i + 2:4 * i + 3, :]
        y1 = pred_ref[0, 4 * i + 3:4 * i + 4, :]
        cx, cy = (x0 + x1) / 2.0, (y0 + y1) / 2.0
        w, h = x1 - x0, y1 - y0
        ts = t[i * nrow:i * nrow + nt, :]
        d = jnp.abs(cx - ts[:, 0:1]) + jnp.abs(cy - ts[:, 1:2])
        d = d + jnp.abs(w - ts[:, 2:3]) + jnp.abs(h - ts[:, 3:4])
        drows.append(d)
        ta = t[i * nrow + nt:(i + 1) * nrow, :]
        da = jnp.abs(a0 - ta[:, 0:1]) + jnp.abs(a1 - ta[:, 1:2])
        da = da + jnp.abs(a2 - ta[:, 2:3]) + jnp.abs(a3 - ta[:, 3:4])
        drows.append(da)
    d = jnp.concatenate(drows, axis=0)
    qio = jax.lax.broadcasted_iota(jnp.int32, (nb * nrow, nq), 1)
    ams = []
    for m in range(_MATCH):
        mn = jnp.min(d, axis=1, keepdims=True)
        am = jnp.min(jnp.where(d == mn, qio, nq), axis=1, keepdims=True)
        ams.append(am)
        if m + 1 < _MATCH:
            d = jnp.where(qio == am, jnp.float32(jnp.inf), d)
    out_ref[0] = jnp.concatenate(ams, axis=1)


def kernel(img_size, pred_boxes, anchor_boxes, tgt_boxes):
    bs, nq = pred_boxes.shape[:2]
    nt = tgt_boxes.shape[1]
    nsc = 5
    nca = nt - 4 * nsc

    anch_t = anchor_boxes.transpose(1, 0)
    tgt_c = _cxcywh(tgt_boxes.reshape(bs * nt, 4) * img_size).reshape(bs, nt, 4)
    tgt_sc = (
        tgt_c[:, nca:, :].transpose(0, 2, 1)
        .reshape(bs, 4, 4, nsc).transpose(0, 2, 1, 3)
    )
    pad = jnp.zeros((bs, 4, 4, _L - nsc), jnp.float32)
    tgt_sc = jnp.concatenate([tgt_sc, pad], axis=-1)

    mesh = plsc.VectorSubcoreMesh(core_axis_name="c", subcore_axis_name="s")
    out_sc = pl.kernel(
        functools.partial(_sc_body, nq, nsc),
        mesh=mesh,
        out_type=jax.ShapeDtypeStruct((bs, 4, _MATCH, _L), jnp.int32),
        scratch_types=[
            pltpu.VMEM((4, nq), jnp.float32),
            pltpu.VMEM((4, _L), jnp.float32),
            pltpu.VMEM((_MATCH, _L), jnp.int32),
        ],
    )(anch_t, tgt_sc)

    nb = 2
    nrow = nt + nca
    ngrid = bs // nb
    pred_tc = pred_boxes.transpose(0, 2, 1).reshape(ngrid, nb * 4, nq)
    tgt_tc = jnp.concatenate([tgt_c, tgt_c[:, :nca, :]], axis=1)
    tgt_tc = tgt_tc.reshape(ngrid, nb * nrow, 4)
    out_tc = pl.pallas_call(
        functools.partial(_tc_body, nq, nt, nb, nca),
        grid=(ngrid,),
        in_specs=[
            pl.BlockSpec((1, nb * 4, nq), lambda g: (g, 0, 0)),
            pl.BlockSpec((4, nq), lambda g: (0, 0)),
            pl.BlockSpec((1, nb * nrow, 4), lambda g: (g, 0, 0)),
        ],
        out_specs=pl.BlockSpec((1, nb * nrow, _MATCH), lambda g: (g, 0, 0)),
        out_shape=jax.ShapeDtypeStruct((ngrid, nb * nrow, _MATCH), jnp.int32),
    )(pred_tc, anch_t, tgt_tc)

    out_tc = (
        out_tc.reshape(ngrid, nb, nrow, _MATCH).transpose(0, 1, 3, 2)
        .reshape(bs, _MATCH, nrow)
    )
    pred_idx = out_tc[:, :, :nt]
    anch_tc = out_tc[:, :, nt:]
    anch_sc = (
        out_sc[:, :, :, :nsc].transpose(0, 2, 1, 3).reshape(bs, _MATCH, 4 * nsc)
    )
    I = jnp.concatenate([pred_idx, anch_tc, anch_sc], axis=2)
    I = I.reshape(bs, _MATCH * 2 * nt)
    j_np = np.tile(np.tile(np.arange(nt, dtype=np.int32), 2), _MATCH)
    J = jnp.asarray(np.tile(j_np[None, :], (bs, 1)))
    return (I, J)


# speedup vs baseline: 1.0089x; 1.0089x over previous
import functools

import jax
import jax.numpy as jnp
import numpy as np
from jax import lax
from jax.experimental import pallas as pl
from jax.experimental.pallas import tpu as pltpu
from jax.experimental.pallas import tpu_sc as plsc

_MATCH = 4
_L = 16

_GATHER_DNUMS = lax.GatherDimensionNumbers(
    offset_dims=(), collapsed_slice_dims=(0,), start_index_map=(0,)
)


def _lane_shuffle(x, perm):
    return lax.gather(
        x,
        perm[:, None],
        _GATHER_DNUMS,
        slice_sizes=(1,),
        mode=lax.GatherScatterMode.PROMISE_IN_BOUNDS,
    )


def _cxcywh(x):
    x0, y0, x1, y1 = jnp.split(x, 4, axis=-1)
    return jnp.concatenate(
        [(x0 + x1) / 2.0, (y0 + y1) / 2.0, x1 - x0, y1 - y0], axis=-1
    )


def _sc_body(nq, ncol, anch_hbm, tgt_hbm, out_hbm, coords_v, tgt_v, out_v):
    nchunk = nq // _L
    cid = lax.axis_index("c")
    sid = lax.axis_index("s")
    wid = sid * 2 + cid
    b = wid // 4
    sub = wid % 4

    pltpu.sync_copy(anch_hbm, coords_v)
    pltpu.sync_copy(tgt_hbm.at[b, sub], tgt_v)

    lane = lax.broadcasted_iota(jnp.int32, (_L,), 0)
    inf = jnp.float32(jnp.inf)
    trow0 = tgt_v[0]
    trow1 = tgt_v[1]
    trow2 = tgt_v[2]
    trow3 = tgt_v[3]

    def per_target(c, acc):
        t0 = trow0[c]
        t1 = trow1[c]
        t2 = trow2[c]
        t3 = trow3[c]

        def scan_chunk(i, carry):
            m0, m1, m2, m3, j0, j1, j2, j3, idx = carry
            s = pl.ds(i * _L, _L)
            v = jnp.abs(coords_v[0, s] - t0)
            v = v + jnp.abs(coords_v[1, s] - t1)
            v = v + jnp.abs(coords_v[2, s] - t2)
            v = v + jnp.abs(coords_v[3, s] - t3)
            b0 = v < m0
            b1 = v < m1
            b2 = v < m2
            b3 = v < m3
            m3 = jnp.where(b3, jnp.where(b2, m2, v), m3)
            j3 = jnp.where(b3, jnp.where(b2, j2, idx), j3)
            m2 = jnp.where(b2, jnp.where(b1, m1, v), m2)
            j2 = jnp.where(b2, jnp.where(b1, j1, idx), j2)
            m1 = jnp.where(b1, jnp.where(b0, m0, v), m1)
            j1 = jnp.where(b1, jnp.where(b0, j0, idx), j1)
            m0 = jnp.where(b0, v, m0)
            j0 = jnp.where(b0, idx, j0)
            return (m0, m1, m2, m3, j0, j1, j2, j3, idx + _L)

        finf = jnp.full((_L,), inf)
        zi = jnp.zeros((_L,), jnp.int32)
        m0, m1, m2, m3, j0, j1, j2, j3, _ = lax.fori_loop(
            0, nchunk, scan_chunk,
            (finf, finf, finf, finf, zi, zi, zi, zi, lane),
        )

        for r in range(_MATCH):
            gv, gj = m0, j0
            for off in (8, 4, 2, 1):
                perm = lane ^ off
                pv = _lane_shuffle(gv, perm)
                pj = _lane_shuffle(gj, perm)
                pick = (pv < gv) | ((pv == gv) & (pj < gj))
                gv = jnp.where(pick, pv, gv)
                gj = jnp.where(pick, pj, gj)
            hit = (m0 == gv) & (j0 == gj)
            m0 = jnp.where(hit, m1, m0)
            j0 = jnp.where(hit, j1, j0)
            m1 = jnp.where(hit, m2, m1)
            j1 = jnp.where(hit, j2, j1)
            m2 = jnp.where(hit, m3, m2)
            j2 = jnp.where(hit, j3, j2)
            m3 = jnp.where(hit, inf, m3)
            acc = tuple(
                jnp.where(lane == c, gj, acc[q]) if q == r else acc[q]
                for q in range(_MATCH)
            )
        return acc

    zacc = jnp.zeros((_L,), jnp.int32)
    acc = (zacc,) * _MATCH
    for c in range(ncol):
        acc = per_target(c, acc)
    for r in range(_MATCH):
        out_v[r] = acc[r]
    pltpu.sync_copy(out_v, out_hbm.at[b, sub])


def _tc_body(nq, nt, nb, nca, pred_ref, anch_ref, tgt_ref, out_ref):
    nrow = nt + nca
    t = tgt_ref[0]
    a0 = anch_ref[0:1, :]
    a1 = anch_ref[1:2, :]
    a2 = anch_ref[2:3, :]
    a3 = anch_ref[3:4, :]
    drows = []
    for i in range(nb):
        x0 = pred_ref[0, 4 * i + 0:4 * i + 1, :]
        y0 = pred_ref[0, 4 * i + 1:4 * i + 2, :]
        x1 = pred_ref[0, 4 * i + 2:4 * i + 3, :]
        y1 = pred_ref[0, 4 * i + 3:4 * i + 4, :]
        cx, cy = (x0 + x1) / 2.0, (y0 + y1) / 2.0
        w, h = x1 - x0, y1 - y0
        ts = t[i * nrow:i * nrow + nt, :]
        d = jnp.abs(cx - ts[:, 0:1]) + jnp.abs(cy - ts[:, 1:2])
        d = d + jnp.abs(w - ts[:, 2:3]) + jnp.abs(h - ts[:, 3:4])
        drows.append(d)
        ta = t[i * nrow + nt:(i + 1) * nrow, :]
        da = jnp.abs(a0 - ta[:, 0:1]) + jnp.abs(a1 - ta[:, 1:2])
        da = da + jnp.abs(a2 - ta[:, 2:3]) + jnp.abs(a3 - ta[:, 3:4])
        drows.append(da)
    d = jnp.concatenate(drows, axis=0)
    qio = jax.lax.broadcasted_iota(jnp.int32, (nb * nrow, nq), 1)
    ams = []
    for m in range(_MATCH):
        mn = jnp.min(d, axis=1, keepdims=True)
        am = jnp.min(jnp.where(d == mn, qio, nq), axis=1, keepdims=True)
        ams.append(am)
        if m + 1 < _MATCH:
            d = jnp.where(qio == am, jnp.float32(jnp.inf), d)
    out_ref[0] = jnp.concatenate(ams, axis=1)


def kernel(img_size, pred_boxes, anchor_boxes, tgt_boxes):
    bs, nq = pred_boxes.shape[:2]
    nt = tgt_boxes.shape[1]
    nsc = 5
    nca = nt - 4 * nsc

    anch_t = anchor_boxes.transpose(1, 0)
    tgt_c = _cxcywh(tgt_boxes.reshape(bs * nt, 4) * img_size).reshape(bs, nt, 4)
    tgt_sc = (
        tgt_c[:, nca:, :].transpose(0, 2, 1)
        .reshape(bs, 4, 4, nsc).transpose(0, 2, 1, 3)
    )
    pad = jnp.zeros((bs, 4, 4, _L - nsc), jnp.float32)
    tgt_sc = jnp.concatenate([tgt_sc, pad], axis=-1)

    mesh = plsc.VectorSubcoreMesh(core_axis_name="c", subcore_axis_name="s")
    out_sc = pl.kernel(
        functools.partial(_sc_body, nq, nsc),
        mesh=mesh,
        out_type=jax.ShapeDtypeStruct((bs, 4, _MATCH, _L), jnp.int32),
        scratch_types=[
            pltpu.VMEM((4, nq), jnp.float32),
            pltpu.VMEM((4, _L), jnp.float32),
            pltpu.VMEM((_MATCH, _L), jnp.int32),
        ],
    )(anch_t, tgt_sc)

    nb = 4
    nrow = nt + nca
    ngrid = bs // nb
    pred_tc = pred_boxes.transpose(0, 2, 1).reshape(ngrid, nb * 4, nq)
    tgt_tc = jnp.concatenate([tgt_c, tgt_c[:, :nca, :]], axis=1)
    tgt_tc = tgt_tc.reshape(ngrid, nb * nrow, 4)
    out_tc = pl.pallas_call(
        functools.partial(_tc_body, nq, nt, nb, nca),
        grid=(ngrid,),
        in_specs=[
            pl.BlockSpec((1, nb * 4, nq), lambda g: (g, 0, 0)),
            pl.BlockSpec((4, nq), lambda g: (0, 0)),
            pl.BlockSpec((1, nb * nrow, 4), lambda g: (g, 0, 0)),
        ],
        out_specs=pl.BlockSpec((1, nb * nrow, _MATCH), lambda g: (g, 0, 0)),
        out_shape=jax.ShapeDtypeStruct((ngrid, nb * nrow, _MATCH), jnp.int32),
    )(pred_tc, anch_t, tgt_tc)

    out_tc = (
        out_tc.reshape(ngrid, nb, nrow, _MATCH).transpose(0, 1, 3, 2)
        .reshape(bs, _MATCH, nrow)
    )
    pred_idx = out_tc[:, :, :nt]
    anch_tc = out_tc[:, :, nt:]
    anch_sc = (
        out_sc[:, :, :, :nsc].transpose(0, 2, 1, 3).reshape(bs, _MATCH, 4 * nsc)
    )
    I = jnp.concatenate([pred_idx, anch_tc, anch_sc], axis=2)
    I = I.reshape(bs, _MATCH * 2 * nt)
    j_np = np.tile(np.tile(np.arange(nt, dtype=np.int32), 2), _MATCH)
    J = jnp.asarray(np.tile(j_np[None, :], (bs, 1)))
    return (I, J)
